# Initial kernel scaffold; baseline (speedup 1.0000x reference)
import jax
import jax.numpy as jnp
from jax.experimental import pallas as pl


def kernel(x, table, W1, b1, W2, b2):
    raise NotImplementedError("write your pallas kernel here")


# speedup vs baseline: 6.6497x; 6.6497x over previous
import functools

import jax
import jax.numpy as jnp
from jax import lax
from jax.experimental import pallas as pl
from jax.experimental.pallas import tpu as pltpu
from jax.experimental.pallas import tpu_sc as plsc

EMBED = 128
L = 20
NCLS = 28
NCORES = 2
NSUB = 16
NWORKERS = NCORES * NSUB
GCHUNK = 128
KBUF = 4


def _sc_gather(table, idx2d, n_rows):
    rows_per_w = n_rows // NWORKERS
    nch = rows_per_w // GCHUNK
    nit = nch // KBUF
    mesh = plsc.VectorSubcoreMesh(core_axis_name="c", subcore_axis_name="s")

    @functools.partial(
        pl.kernel,
        out_type=jax.ShapeDtypeStruct((n_rows, EMBED), jnp.float32),
        mesh=mesh,
        scratch_types=[pltpu.VMEM((nch, GCHUNK), jnp.int32)]
        + [pltpu.VMEM((GCHUNK, EMBED), jnp.float32) for _ in range(KBUF)]
        + [pltpu.SemaphoreType.DMA],
    )
    def k(table_hbm, idx_hbm, out_hbm, idx_v, *bufs_sem):
        bufs, sem = bufs_sem[:KBUF], bufs_sem[KBUF]
        w = lax.axis_index("s") * NCORES + lax.axis_index("c")
        pltpu.sync_copy(idx_hbm.at[pl.ds(w * nch, nch)], idx_v)

        @pl.loop(0, nit)
        def _(jj):
            base_c = jj * KBUF
            cps = [
                pltpu.async_copy(table_hbm.at[idx_v.at[base_c + p]], bufs[p], sem)
                for p in range(KBUF)
            ]
            for p in range(KBUF):
                cps[p].wait()
                pltpu.sync_copy(
                    bufs[p],
                    out_hbm.at[pl.ds(w * rows_per_w + (base_c + p) * GCHUNK, GCHUNK)],
                )

    return k(table, idx2d)


def _tc_mlp(g3, W1, b1, W2, b2, batch):
    BB = 512

    def body(g_ref, w1_ref, b1_ref, w2_ref, b2_ref, o_ref):
        pooled = jnp.sum(g_ref[...], axis=0) * (1.0 / L)
        h = jnp.maximum(jnp.dot(pooled, w1_ref[...],
                                preferred_element_type=jnp.float32) + b1_ref[...], 0.0)
        o_ref[...] = jnp.dot(h, w2_ref[...],
                             preferred_element_type=jnp.float32) + b2_ref[...]

    return pl.pallas_call(
        body,
        grid=(batch // BB,),
        in_specs=[
            pl.BlockSpec((L, BB, EMBED), lambda i: (0, i, 0)),
            pl.BlockSpec((EMBED, W1.shape[1]), lambda i: (0, 0)),
            pl.BlockSpec((1, W1.shape[1]), lambda i: (0, 0)),
            pl.BlockSpec((W1.shape[1], NCLS), lambda i: (0, 0)),
            pl.BlockSpec((1, NCLS), lambda i: (0, 0)),
        ],
        out_specs=pl.BlockSpec((BB, NCLS), lambda i: (i, 0)),
        out_shape=jax.ShapeDtypeStruct((batch, NCLS), jnp.float32),
    )(g3, W1, b1.reshape(1, -1), W2, b2.reshape(1, -1))


def kernel(x, table, W1, b1, W2, b2):
    batch, seq = x.shape
    n_rows = batch * seq
    idx2d = x.astype(jnp.int32).T.reshape(n_rows // GCHUNK, GCHUNK)
    g = _sc_gather(table, idx2d, n_rows)
    g3 = g.reshape(seq, batch, EMBED)
    return _tc_mlp(g3, W1, b1, W2, b2, batch)
